# baseline (speedup 1.0000x reference)
import functools

import jax
import jax.numpy as jnp
from jax import lax
from jax.experimental import pallas as pl
from jax.experimental.pallas import tpu as pltpu
from jax.experimental.pallas import tpu_sc as plsc

R, C = 128, 32768
L = 16
NC, NS = 2, 16
NW = NC * NS
SLAB = 8
NQ = 4
QCOL = C // NQ
CHUNK = 2048
NCH = QCOL // CHUNK
RSC = 64
BLK = 2048
NBLK = C // BLK


def _sc_body(x_hbm, vals_hbm, idxs_hbm, cbuf0, cbuf1, stgv, stgi, sem_in):
    c = lax.axis_index("c")
    s = lax.axis_index("s")
    wid = c * NS + s
    slab = c * (NS // NQ) + s // NQ
    q = s % NQ
    rr0 = pl.multiple_of(slab * SLAB, SLAB)
    cc0 = pl.multiple_of(q * QCOL, 128)
    lanes = lax.iota(jnp.int32, L)
    lane0 = lanes == 0
    cbufs = [cbuf0, cbuf1]

    def chunk_src(ch):
        col = pl.multiple_of(cc0 + ch * CHUNK, 128)
        return x_hbm.at[pl.ds(rr0, SLAB), pl.ds(col, CHUNK)]

    cp_in = pltpu.async_copy(chunk_src(0), cbufs[0], sem_in)

    neg_inf = jnp.full((L,), -jnp.inf, jnp.float32)
    vmaxs = [neg_inf] * SLAB
    vidxs = [jnp.zeros((L,), jnp.int32)] * SLAB
    for ch in range(NCH):
        cp_in.wait()
        if ch + 1 < NCH:
            cp_in = pltpu.async_copy(
                chunk_src(ch + 1), cbufs[(ch + 1) % 2], sem_in
            )
        buf = cbufs[ch % 2]
        cidx0 = (cc0 + ch * CHUNK) + lanes

        def sbody(i, carry):
            vm, vi, cidx = carry
            nvm, nvi = [], []
            for r in range(SLAB):
                v = buf[r, pl.ds(i * L, L)]
                m = v > vm[r]
                nvm.append(jnp.where(m, v, vm[r]))
                nvi.append(jnp.where(m, cidx, vi[r]))
            return tuple(nvm), tuple(nvi), cidx + L

        vm, vi, _ = lax.fori_loop(
            0, CHUNK // L, sbody, (tuple(vmaxs), tuple(vidxs), cidx0)
        )
        vmaxs, vidxs = list(vm), list(vi)

    for r in range(SLAB):
        gmax = jnp.max(vmaxs[r])
        idx = jnp.min(jnp.where(vmaxs[r] == gmax, vidxs[r], jnp.int32(C)))
        rv = jnp.full((L,), r, jnp.int32)
        plsc.store_scatter(stgv, [rv], jnp.full((L,), 0.0, jnp.float32) + gmax,
                           mask=lane0)
        plsc.store_scatter(stgi, [rv], jnp.zeros((L,), jnp.int32) + idx,
                           mask=lane0)

    pltpu.sync_copy(stgv, vals_hbm.at[wid, pl.ds(0, L)])
    pltpu.sync_copy(stgi, idxs_hbm.at[wid, pl.ds(0, L)])


def _tc_argmax_body(x_ref, val_ref, idx_ref, vm, vi):
    i = pl.program_id(0)
    xb = x_ref[...]
    bmax = jnp.max(xb, axis=1, keepdims=True)
    iot = lax.broadcasted_iota(jnp.int32, (R - RSC, BLK), 1) + i * BLK
    bidx = jnp.min(
        jnp.where(xb == bmax, iot, jnp.int32(C)), axis=1, keepdims=True
    )

    @pl.when(i == 0)
    def _():
        vm[...] = bmax
        vi[...] = bidx

    @pl.when(i > 0)
    def _():
        pv, pi_ = vm[...], vi[...]
        better = (bmax > pv) | ((bmax == pv) & (bidx < pi_))
        vm[...] = jnp.where(better, bmax, pv)
        vi[...] = jnp.where(better, bidx, pi_)

    val_ref[...] = vm[...]
    idx_ref[...] = vi[...]


def _tc_expand_body(cv_ref, ci_ref, out_ref):
    i = pl.program_id(0)
    cv = cv_ref[...]
    ci = ci_ref[...]
    bv, bi = cv[:, 0:1], ci[:, 0:1]
    for k in range(1, NQ):
        nv, ni = cv[:, k : k + 1], ci[:, k : k + 1]
        better = (nv > bv) | ((nv == bv) & (ni < bi))
        bv = jnp.where(better, nv, bv)
        bi = jnp.where(better, ni, bi)
    iot = lax.broadcasted_iota(jnp.int32, (R, BLK), 1) + i * BLK
    out_ref[...] = (iot == bi).astype(jnp.float32)


@jax.jit
def kernel(x):
    mesh = plsc.VectorSubcoreMesh(
        core_axis_name="c", subcore_axis_name="s", num_cores=NC, num_subcores=NS
    )
    sc = functools.partial(
        pl.kernel,
        mesh=mesh,
        out_type=(
            jax.ShapeDtypeStruct((NW, 128), jnp.float32),
            jax.ShapeDtypeStruct((NW, 128), jnp.int32),
        ),
        scratch_types=[
            pltpu.VMEM((SLAB, CHUNK), jnp.float32),
            pltpu.VMEM((SLAB, CHUNK), jnp.float32),
            pltpu.VMEM((L,), jnp.float32),
            pltpu.VMEM((L,), jnp.int32),
            pltpu.SemaphoreType.DMA,
        ],
        compiler_params=pltpu.CompilerParams(needs_layout_passes=False),
    )(_sc_body)
    sv, si = sc(x)

    tv, ti = pl.pallas_call(
        _tc_argmax_body,
        grid=(NBLK,),
        in_specs=[
            pl.BlockSpec((R - RSC, BLK), lambda i: (1, i)),
        ],
        out_specs=[
            pl.BlockSpec((R - RSC, 1), lambda i: (0, 0)),
            pl.BlockSpec((R - RSC, 1), lambda i: (0, 0)),
        ],
        out_shape=[
            jax.ShapeDtypeStruct((R - RSC, 1), jnp.float32),
            jax.ShapeDtypeStruct((R - RSC, 1), jnp.int32),
        ],
        scratch_shapes=[
            pltpu.VMEM((R - RSC, 1), jnp.float32),
            pltpu.VMEM((R - RSC, 1), jnp.int32),
        ],
    )(x)

    a_val = sv[:, :SLAB].reshape(NC, NS // NQ, NQ, SLAB)
    a_val = jnp.transpose(a_val, (0, 1, 3, 2)).reshape(RSC, NQ)
    a_idx = si[:, :SLAB].reshape(NC, NS // NQ, NQ, SLAB)
    a_idx = jnp.transpose(a_idx, (0, 1, 3, 2)).reshape(RSC, NQ)
    b_val = jnp.concatenate(
        [tv, jnp.full((R - RSC, NQ - 1), -jnp.inf, jnp.float32)], axis=1
    )
    b_idx = jnp.concatenate(
        [ti, jnp.zeros((R - RSC, NQ - 1), jnp.int32)], axis=1
    )
    cand_val = jnp.concatenate([a_val, b_val], axis=0)
    cand_idx = jnp.concatenate([a_idx, b_idx], axis=0)

    out = pl.pallas_call(
        _tc_expand_body,
        grid=(NBLK,),
        in_specs=[
            pl.BlockSpec((R, NQ), lambda i: (0, 0)),
            pl.BlockSpec((R, NQ), lambda i: (0, 0)),
        ],
        out_specs=pl.BlockSpec((R, BLK), lambda i: (0, i)),
        out_shape=jax.ShapeDtypeStruct((R, C), jnp.float32),
    )(cand_val, cand_idx)
    return out


# speedup vs baseline: 1.3725x; 1.3725x over previous
import functools

import jax
import jax.numpy as jnp
from jax import lax
from jax.experimental import pallas as pl
from jax.experimental.pallas import tpu as pltpu
from jax.experimental.pallas import tpu_sc as plsc

R, C = 128, 32768
L = 16
NC, NS = 2, 16
NW = NC * NS
ROWS_PER_W = R // NW
U = 8
STRIDE = U * L


def _scan_row(inbuf, lanes):
    neg_inf = jnp.full((L,), -jnp.inf, jnp.float32)
    zero_i = jnp.zeros((L,), jnp.int32)

    def body(i, carry):
        vmaxs, vidxs, cidx = carry
        new_vmaxs = []
        new_vidxs = []
        for a in range(U):
            v = inbuf[pl.ds(i * STRIDE + a * L, L)]
            m = v > vmaxs[a]
            new_vmaxs.append(jnp.where(m, v, vmaxs[a]))
            new_vidxs.append(jnp.where(m, cidx, vidxs[a]))
        return tuple(new_vmaxs), tuple(new_vidxs), cidx + STRIDE

    vmaxs, vidxs, _ = lax.fori_loop(
        0,
        C // STRIDE,
        body,
        ((neg_inf,) * U, (zero_i,) * U, zero_i),
    )
    vidxs = [vidxs[a] + (a * L) + lanes for a in range(U)]
    vmaxs = list(vmaxs)
    n = U
    while n > 1:
        n //= 2
        for a in range(n):
            ov, oi = vmaxs[a + n], vidxs[a + n]
            better = (ov > vmaxs[a]) | ((ov == vmaxs[a]) & (oi < vidxs[a]))
            vmaxs[a] = jnp.where(better, ov, vmaxs[a])
            vidxs[a] = jnp.where(better, oi, vidxs[a])
    gmax = jnp.max(vmaxs[0])
    cand = jnp.where(vmaxs[0] == gmax, vidxs[0], jnp.int32(C))
    return jnp.min(cand)


def _body(
    x_hbm, out_hbm, inbuf0, inbuf1, zbuf, pbuf, sem_in, sem_z, sem_p
):
    wid = lax.axis_index("s") * NC + lax.axis_index("c")
    lanes = lax.iota(jnp.int32, L)
    zeros_v = jnp.zeros((L,), jnp.float32)
    ones_v = jnp.ones((L,), jnp.float32)
    lane0 = lanes == 0
    r0 = wid * ROWS_PER_W
    inbufs = [inbuf0, inbuf1]

    cp_in = pltpu.async_copy(x_hbm.at[r0], inbufs[0], sem_in)

    def zbody(i, _):
        for a in range(U):
            zbuf[pl.ds(i * STRIDE + a * L, L)] = zeros_v
        return 0

    lax.fori_loop(0, C // STRIDE, zbody, 0)
    for k in range(ROWS_PER_W):
        pbuf[pl.ds(k * L, L)] = zeros_v

    zcps = [
        pltpu.async_copy(zbuf, out_hbm.at[r0 + k], sem_z)
        for k in range(ROWS_PER_W)
    ]

    segs = []
    for k in range(ROWS_PER_W):
        cp_in.wait()
        if k + 1 < ROWS_PER_W:
            cp_in = pltpu.async_copy(
                x_hbm.at[r0 + k + 1], inbufs[(k + 1) % 2], sem_in
            )
        idx = _scan_row(inbufs[k % 2], lanes)
        seg = pl.multiple_of((idx // L) * L, L)
        off = jnp.full((L,), k * L, jnp.int32) + (idx - seg)
        plsc.store_scatter(pbuf, [off], ones_v, mask=lane0)
        segs.append(seg)

    for z in zcps:
        z.wait()
    pcps = [
        pltpu.async_copy(
            pbuf.at[pl.ds(k * L, L)],
            out_hbm.at[r0 + k, pl.ds(segs[k], L)],
            sem_p,
        )
        for k in range(ROWS_PER_W)
    ]
    for p in pcps:
        p.wait()


@jax.jit
def kernel(x):
    mesh = plsc.VectorSubcoreMesh(
        core_axis_name="c", subcore_axis_name="s", num_cores=NC, num_subcores=NS
    )
    f = functools.partial(
        pl.kernel,
        mesh=mesh,
        out_type=jax.ShapeDtypeStruct((R, C), jnp.float32),
        scratch_types=[
            pltpu.VMEM((C,), jnp.float32),
            pltpu.VMEM((C,), jnp.float32),
            pltpu.VMEM((C,), jnp.float32),
            pltpu.VMEM((ROWS_PER_W * L,), jnp.float32),
            pltpu.SemaphoreType.DMA,
            pltpu.SemaphoreType.DMA,
            pltpu.SemaphoreType.DMA,
        ],
        compiler_params=pltpu.CompilerParams(needs_layout_passes=False),
    )(_body)
    return f(x)
